# baseline (speedup 1.0000x reference)
import functools

import numpy as np
import jax
import jax.numpy as jnp
from jax import lax
from jax.experimental import pallas as pl
from jax.experimental.pallas import tpu as pltpu
from jax.experimental.pallas import tpu_sc as plsc

DEG = 8
_LANES = 16
_NC = 2
_NS = 16
_NW = _NC * _NS

_CHEB2G = np.array(
    [
        [1.0, 0.0, 0.0, 0.0],
        [0.0, -1.0, 0.0, 0.0],
        [-1.0, 0.0, 2.0, 0.0],
        [0.0, 3.0, 0.0, -4.0],
    ],
    dtype=np.float32,
)


def _chunk_nodes(f):
    return min(128 // DEG, max(1, 8192 // f))


def _sc_grid(n, f):
    t = _chunk_nodes(f)
    nchunk = -(-n // t)
    rounds = -(-nchunk // _NW)
    return t, rounds, rounds * _NW * t


@functools.cache
def _gather_agg_fn(n, f):
    t, rounds, n_pad = _sc_grid(n, f)
    e_chunk = t * DEG

    mesh = plsc.VectorSubcoreMesh(core_axis_name="c", subcore_axis_name="s")

    def body(feat_hbm, src_hbm, w_hbm, out_hbm, idx_v, w_v, rows_v, acc_v, sem):
        cid = lax.axis_index("c")
        sid = lax.axis_index("s")
        wid = sid * _NC + cid

        def round_body(k, carry):
            c = wid + k * _NW
            base_e = c * e_chunk
            pltpu.sync_copy(src_hbm.at[pl.ds(base_e, e_chunk)], idx_v)
            pltpu.sync_copy(w_hbm.at[pl.ds(base_e, e_chunk)], w_v)
            pltpu.async_copy(feat_hbm.at[idx_v], rows_v, sem).wait()

            def node_body(i, carry2):
                e0 = i * DEG
                ws = [w_v[e0 + j, :] for j in range(DEG)]
                for fo in range(f // _LANES):
                    sl = pl.ds(fo * _LANES, _LANES)
                    acc = ws[0] * rows_v[e0, sl]
                    for j in range(1, DEG):
                        acc = acc + ws[j] * rows_v[e0 + j, sl]
                    acc_v[i, sl] = acc
                return carry2

            lax.fori_loop(0, t, node_body, 0)
            pltpu.sync_copy(acc_v, out_hbm.at[pl.ds(c * t, t)])
            return carry

        lax.fori_loop(0, rounds, round_body, 0)

    return pl.kernel(
        body,
        out_type=jax.ShapeDtypeStruct((n_pad, f), jnp.float32),
        mesh=mesh,
        compiler_params=pltpu.CompilerParams(use_tc_tiling_on_sc=False),
        scratch_types=[
            pltpu.VMEM((e_chunk,), jnp.int32),
            pltpu.VMEM((e_chunk, _LANES), jnp.float32),
            pltpu.VMEM((e_chunk, f), jnp.float32),
            pltpu.VMEM((t, f), jnp.float32),
            pltpu.SemaphoreType.DMA,
        ],
    )


def _pick_tm(m):
    for tm in (512, 384, 288, 256, 128, 96, 72, 64, 32, 16, 8):
        if m % tm == 0:
            return tm
    return m


@functools.cache
def _mix_fn(m, cin, o, b, skip_mode, cs, relu, pool):
    tm = _pick_tm(m)
    grid = (m // tm,)
    feat_spec = pl.BlockSpec((tm, cin), lambda i: (i, 0))
    full = lambda shape: pl.BlockSpec(shape, lambda i: tuple(0 for _ in shape))

    in_specs = [feat_spec] * 4 + [full((4 * cin, o)), full((1, o))]
    if skip_mode == "proj":
        in_specs += [pl.BlockSpec((tm, cs), lambda i: (i, 0)), full((cs, o))]
    elif skip_mode == "id":
        in_specs += [pl.BlockSpec((tm, o), lambda i: (i, 0))]

    out_shape = [jax.ShapeDtypeStruct((m, o), jnp.float32)]
    out_specs = [pl.BlockSpec((tm, o), lambda i: (i, 0))]
    if pool:
        out_shape.append(jax.ShapeDtypeStruct((m // 4, o), jnp.float32))
        out_specs.append(pl.BlockSpec((tm // 4, o), lambda i: (i, 0)))

    def body(*refs):
        refs = list(refs)
        g0, g1, g2, g3, w_ref, b_ref = refs[:6]
        refs = refs[6:]
        gcat = jnp.concatenate([g0[...], g1[...], g2[...], g3[...]], axis=1)
        acc = jnp.dot(gcat, w_ref[...], preferred_element_type=jnp.float32,
                      precision=lax.Precision.HIGHEST)
        acc = acc + b_ref[...]
        if skip_mode == "proj":
            s_ref, sw_ref = refs[:2]
            refs = refs[2:]
            acc = acc + jnp.dot(
                s_ref[...], sw_ref[...], preferred_element_type=jnp.float32,
                precision=lax.Precision.HIGHEST,
            )
        elif skip_mode == "id":
            s_ref = refs[0]
            refs = refs[1:]
            acc = acc + s_ref[...]
        if relu:
            acc = jnp.maximum(acc, 0.0)
        out_ref = refs[0]
        out_ref[...] = acc
        if pool:
            pool_ref = refs[1]
            grp = acc.reshape(tm // (4 * b), 4, b, o)
            pool_ref[...] = jnp.max(grp, axis=1).reshape(tm // 4, o)

    return pl.pallas_call(
        body,
        grid=grid,
        in_specs=in_specs,
        out_specs=out_specs if len(out_specs) > 1 else out_specs[0],
        out_shape=out_shape if len(out_shape) > 1 else out_shape[0],
    )


def _cheb_conv(rows, n, b, graph, cp, relu, skip_rows=None, skip_w=None,
               skip_id=False, pool=False):
    src, _dst, w = graph
    cin = rows.shape[-1]
    o = cp["W"].shape[-1]
    f = b * cin
    agg = _gather_agg_fn(n, f)
    t, _r, n_pad = _sc_grid(n, f)
    if n_pad > n:
        pad = ((0, (n_pad - n) * DEG),)
        src = jnp.pad(src, pad)
        w = jnp.pad(w, pad)
    w2 = jnp.broadcast_to(w[:, None], (w.shape[0], _LANES))

    def pad_rows(g):
        return (jnp.pad(g, ((0, (n_pad - n)), (0, 0)))
                if n_pad > n else g)

    def hop(g):
        return agg(g, src, w2)

    g0p = pad_rows(rows.reshape(n, f))
    g1p = hop(g0p)
    g2p = hop(g1p)
    g3p = hop(g2p)
    g0 = rows
    g1 = g1p[: n].reshape(n * b, cin) if n_pad > n else g1p.reshape(n * b, cin)
    g2 = g2p[: n].reshape(n * b, cin) if n_pad > n else g2p.reshape(n * b, cin)
    g3 = g3p[: n].reshape(n * b, cin) if n_pad > n else g3p.reshape(n * b, cin)

    wg = jnp.einsum("kg,kco->gco", _CHEB2G, cp["W"]).reshape(4 * cin, o)
    bias = cp["b"].reshape(1, o)

    if skip_rows is not None and skip_w is not None:
        mode, cs = "proj", skip_rows.shape[-1]
        args = (g0, g1, g2, g3, wg, bias, skip_rows, skip_w)
    elif skip_id:
        mode, cs = "id", 0
        args = (g0, g1, g2, g3, wg, bias, skip_rows)
    else:
        mode, cs = "none", 0
        args = (g0, g1, g2, g3, wg, bias)

    return _mix_fn(n * b, cin, o, b, mode, cs, relu, pool)(*args)


def _res_block(rows, n, b, graph, p, pool):
    h = _cheb_conv(rows, n, b, graph, p["c1"], relu=True)
    return _cheb_conv(
        h, n, b, graph, p["c2"], relu=True,
        skip_rows=rows,
        skip_w=p.get("skip"),
        skip_id="skip" not in p,
        pool=pool,
    )


def kernel(x, params, graphs):
    b, cin, n5 = x.shape
    rows = jnp.transpose(x, (2, 0, 1)).reshape(n5 * b, cin)
    sizes = [g[0].shape[0] // DEG for g in graphs]

    h = _cheb_conv(rows, n5, b, graphs[5], params["conv"], relu=True)
    e5, p5 = _res_block(h, sizes[5], b, graphs[5], params["block5"], pool=True)
    e4, p4 = _res_block(p5, sizes[4], b, graphs[4], params["block4"], pool=True)
    e3, p3 = _res_block(p4, sizes[3], b, graphs[3], params["block3"], pool=True)
    e2, p2 = _res_block(p3, sizes[2], b, graphs[2], params["block2"], pool=True)
    e1, p1 = _res_block(p2, sizes[1], b, graphs[1], params["block1"], pool=True)
    e0 = _res_block(p1, sizes[0], b, graphs[0], params["block0"], pool=False)

    outs = []
    for e, n in zip((e0, e1, e2, e3, e4, e5), sizes):
        c = e.shape[-1]
        outs.append(jnp.transpose(e.reshape(n, b, c), (1, 2, 0)))
    return tuple(outs)


# speedup vs baseline: 1.8881x; 1.8881x over previous
import functools

import numpy as np
import jax
import jax.numpy as jnp
from jax import lax
from jax.experimental import pallas as pl
from jax.experimental.pallas import tpu as pltpu
from jax.experimental.pallas import tpu_sc as plsc

DEG = 8
_LANES = 16
_NC = 2
_NS = 16
_NW = _NC * _NS

_CHEB2G = np.array(
    [
        [1.0, 0.0, 0.0, 0.0],
        [0.0, -1.0, 0.0, 0.0],
        [-1.0, 0.0, 2.0, 0.0],
        [0.0, 3.0, 0.0, -4.0],
    ],
    dtype=np.float32,
)


def _chunk_nodes(n, f):
    t = min(128 // DEG, max(1, 8192 // f))
    while n % t:
        t -= 1
    return t


@functools.cache
def _gather_agg_fn(n, f):
    t = _chunk_nodes(n, f)
    e_chunk = t * DEG
    nchunk = n // t
    q, r = divmod(nchunk, _NW)
    rounds_max = q + (1 if r else 0)
    e_wk = rounds_max * e_chunk

    mesh = plsc.VectorSubcoreMesh(core_axis_name="c", subcore_axis_name="s")

    def body(feat_hbm, src_hbm, w_hbm, out_hbm, idx_v, w_v, rows_v, acc_v, sem):
        cid = lax.axis_index("c")
        sid = lax.axis_index("s")
        wid = sid * _NC + cid
        lo = wid * q + jnp.minimum(wid, r)
        cnt = q + jnp.where(wid < r, 1, 0)
        base = jnp.minimum(lo, nchunk - rounds_max)
        pltpu.sync_copy(src_hbm.at[pl.ds(base, rounds_max)], idx_v)
        pltpu.sync_copy(w_hbm.at[pl.ds(base * e_chunk, e_wk)], w_v)
        off = lo - base

        def round_body(k, carry):
            @pl.when(k < cnt)
            def _():
                kk = off + k
                pltpu.async_copy(feat_hbm.at[idx_v.at[kk]], rows_v, sem).wait()
                ew0 = kk * e_chunk

                def node_body(i, carry2):
                    e0 = i * DEG
                    ws = [w_v[ew0 + e0 + j, :] for j in range(DEG)]
                    for fo in range(f // _LANES):
                        sl = pl.ds(fo * _LANES, _LANES)
                        acc = ws[0] * rows_v[e0, sl]
                        for j in range(1, DEG):
                            acc = acc + ws[j] * rows_v[e0 + j, sl]
                        acc_v[i, sl] = acc
                    return carry2

                lax.fori_loop(0, t, node_body, 0)
                pltpu.sync_copy(acc_v, out_hbm.at[pl.ds((lo + k) * t, t)])

            return carry

        lax.fori_loop(0, rounds_max, round_body, 0)

    return pl.kernel(
        body,
        out_type=jax.ShapeDtypeStruct((n, f), jnp.float32),
        mesh=mesh,
        compiler_params=pltpu.CompilerParams(use_tc_tiling_on_sc=False),
        scratch_types=[
            pltpu.VMEM((rounds_max, e_chunk), jnp.int32),
            pltpu.VMEM((e_wk, _LANES), jnp.float32),
            pltpu.VMEM((e_chunk, f), jnp.float32),
            pltpu.VMEM((t, f), jnp.float32),
            pltpu.SemaphoreType.DMA,
        ],
    )


def _pick_tm(m):
    for tm in (512, 384, 288, 256, 128, 96, 72, 64, 32, 16, 8):
        if m % tm == 0:
            return tm
    return m


@functools.cache
def _mix_fn(m, cin, o, b, skip_mode, cs, relu, pool):
    tm = _pick_tm(m)
    grid = (m // tm,)
    feat_spec = pl.BlockSpec((tm, cin), lambda i: (i, 0))
    full = lambda shape: pl.BlockSpec(shape, lambda i: tuple(0 for _ in shape))

    in_specs = [feat_spec] * 4 + [full((4 * cin, o)), full((1, o))]
    if skip_mode == "proj":
        in_specs += [pl.BlockSpec((tm, cs), lambda i: (i, 0)), full((cs, o))]
    elif skip_mode == "id":
        in_specs += [pl.BlockSpec((tm, o), lambda i: (i, 0))]

    out_shape = [jax.ShapeDtypeStruct((m, o), jnp.float32)]
    out_specs = [pl.BlockSpec((tm, o), lambda i: (i, 0))]
    if pool:
        out_shape.append(jax.ShapeDtypeStruct((m // 4, o), jnp.float32))
        out_specs.append(pl.BlockSpec((tm // 4, o), lambda i: (i, 0)))

    def body(*refs):
        refs = list(refs)
        g0, g1, g2, g3, w_ref, b_ref = refs[:6]
        refs = refs[6:]
        gcat = jnp.concatenate([g0[...], g1[...], g2[...], g3[...]], axis=1)
        acc = jnp.dot(gcat, w_ref[...], preferred_element_type=jnp.float32,
                      precision=lax.Precision.HIGHEST)
        acc = acc + b_ref[...]
        if skip_mode == "proj":
            s_ref, sw_ref = refs[:2]
            refs = refs[2:]
            acc = acc + jnp.dot(
                s_ref[...], sw_ref[...], preferred_element_type=jnp.float32,
                precision=lax.Precision.HIGHEST,
            )
        elif skip_mode == "id":
            s_ref = refs[0]
            refs = refs[1:]
            acc = acc + s_ref[...]
        if relu:
            acc = jnp.maximum(acc, 0.0)
        out_ref = refs[0]
        out_ref[...] = acc
        if pool:
            pool_ref = refs[1]
            grp = acc.reshape(tm // (4 * b), 4, b, o)
            pool_ref[...] = jnp.max(grp, axis=1).reshape(tm // 4, o)

    return pl.pallas_call(
        body,
        grid=grid,
        in_specs=in_specs,
        out_specs=out_specs if len(out_specs) > 1 else out_specs[0],
        out_shape=out_shape if len(out_shape) > 1 else out_shape[0],
    )


def _cheb_conv(rows, n, b, graph, cp, relu, skip_rows=None, skip_w=None,
               skip_id=False, pool=False):
    src, _dst, w = graph
    cin = rows.shape[-1]
    o = cp["W"].shape[-1]
    f = b * cin
    agg = _gather_agg_fn(n, f)
    t = _chunk_nodes(n, f)
    src = src.reshape(n // t, t * DEG)
    w2 = jnp.broadcast_to(w[:, None], (w.shape[0], _LANES))

    def hop(g):
        return agg(g, src, w2)

    g0p = rows.reshape(n, f)
    g1p = hop(g0p)
    g2p = hop(g1p)
    g3p = hop(g2p)
    g0 = rows
    g1 = g1p.reshape(n * b, cin)
    g2 = g2p.reshape(n * b, cin)
    g3 = g3p.reshape(n * b, cin)

    wg = jnp.einsum("kg,kco->gco", _CHEB2G, cp["W"]).reshape(4 * cin, o)
    bias = cp["b"].reshape(1, o)

    if skip_rows is not None and skip_w is not None:
        mode, cs = "proj", skip_rows.shape[-1]
        args = (g0, g1, g2, g3, wg, bias, skip_rows, skip_w)
    elif skip_id:
        mode, cs = "id", 0
        args = (g0, g1, g2, g3, wg, bias, skip_rows)
    else:
        mode, cs = "none", 0
        args = (g0, g1, g2, g3, wg, bias)

    return _mix_fn(n * b, cin, o, b, mode, cs, relu, pool)(*args)


def _res_block(rows, n, b, graph, p, pool):
    h = _cheb_conv(rows, n, b, graph, p["c1"], relu=True)
    return _cheb_conv(
        h, n, b, graph, p["c2"], relu=True,
        skip_rows=rows,
        skip_w=p.get("skip"),
        skip_id="skip" not in p,
        pool=pool,
    )


def kernel(x, params, graphs):
    b, cin, n5 = x.shape
    rows = jnp.transpose(x, (2, 0, 1)).reshape(n5 * b, cin)
    sizes = [g[0].shape[0] // DEG for g in graphs]

    h = _cheb_conv(rows, n5, b, graphs[5], params["conv"], relu=True)
    e5, p5 = _res_block(h, sizes[5], b, graphs[5], params["block5"], pool=True)
    e4, p4 = _res_block(p5, sizes[4], b, graphs[4], params["block4"], pool=True)
    e3, p3 = _res_block(p4, sizes[3], b, graphs[3], params["block3"], pool=True)
    e2, p2 = _res_block(p3, sizes[2], b, graphs[2], params["block2"], pool=True)
    e1, p1 = _res_block(p2, sizes[1], b, graphs[1], params["block1"], pool=True)
    e0 = _res_block(p1, sizes[0], b, graphs[0], params["block0"], pool=False)

    outs = []
    for e, n in zip((e0, e1, e2, e3, e4, e5), sizes):
        c = e.shape[-1]
        outs.append(jnp.transpose(e.reshape(n, b, c), (1, 2, 0)))
    return tuple(outs)
